# baseline (speedup 1.0000x reference)
import functools
import math

import jax
import jax.numpy as jnp
import numpy as np
from jax.experimental import pallas as pl
from jax.experimental.pallas import tpu as pltpu

V = 32000
D = 512
H = 8
NE = 8
DFF = 1024
DH = D // H
_SQRT_D = math.sqrt(D)
_ATTN_SCALE = 1.0 / math.sqrt(DH)
_EMB_ROWS = 8


def _pe_table(S):
    pos = np.arange(S)[:, None].astype(np.float32)
    div = np.exp(np.arange(0, D, 2).astype(np.float32) * (-math.log(10000.0) / D))
    pe = np.zeros((S, D), dtype=np.float32)
    pe[:, 0::2] = np.sin(pos * div)
    pe[:, 1::2] = np.cos(pos * div)
    return jnp.asarray(pe)


def _embed_body(idx_ref, *refs):
    del idx_ref
    pe_ref = refs[_EMB_ROWS]
    out_ref = refs[_EMB_ROWS + 1]
    rows = jnp.concatenate([refs[j][0] for j in range(_EMB_ROWS)], axis=0)
    out_ref[...] = rows * _SQRT_D + pe_ref[...]


def _embed_row_map(j, t, idx_ref):
    return (idx_ref[t * _EMB_ROWS + j], 0, 0)


def _embed(table, idx, pe):
    T = idx.shape[0]
    R = _EMB_ROWS
    table = table.reshape(V, 1, D)
    table_specs = [
        pl.BlockSpec((1, 1, D), functools.partial(_embed_row_map, j)) for j in range(R)
    ]
    spec = pltpu.PrefetchScalarGridSpec(
        num_scalar_prefetch=1,
        grid=(T // R,),
        in_specs=table_specs + [pl.BlockSpec((R, D), lambda t, idx_ref: (t, 0))],
        out_specs=pl.BlockSpec((R, D), lambda t, idx_ref: (t, 0)),
    )
    return pl.pallas_call(
        _embed_body,
        grid_spec=spec,
        out_shape=jax.ShapeDtypeStruct((T, D), jnp.float32),
    )(idx, *([table] * R), pe)


def _mm_bias_body(a_ref, w_ref, b_ref, o_ref):
    o_ref[...] = (
        jnp.dot(a_ref[...], w_ref[...], preferred_element_type=jnp.float32)
        + b_ref[...]
    )


def _mm_bias(a, w, b):
    M, _ = a.shape
    N = w.shape[1]
    return pl.pallas_call(
        _mm_bias_body,
        out_shape=jax.ShapeDtypeStruct((M, N), jnp.float32),
    )(a, w, b.reshape(1, N))


def _mm_bias_blocked(a, w, b, nb):
    M, K = a.shape
    N = w.shape[1]
    return pl.pallas_call(
        _mm_bias_body,
        grid=(N // nb,),
        in_specs=[
            pl.BlockSpec((M, K), lambda j: (0, 0)),
            pl.BlockSpec((K, nb), lambda j: (0, j)),
            pl.BlockSpec((1, nb), lambda j: (0, j)),
        ],
        out_specs=pl.BlockSpec((M, nb), lambda j: (0, j)),
        out_shape=jax.ShapeDtypeStruct((M, N), jnp.float32),
    )(a, w, b.reshape(1, N))


def _mha_body(xq_ref, xkv_ref, wq_ref, bq_ref, wk_ref, bk_ref, wv_ref, bv_ref,
              wo_ref, bo_ref, g_ref, bb_ref, o_ref, acc_ref):
    h = pl.program_id(0)
    xq = xq_ref[...]
    xkv = xkv_ref[...]
    q = jnp.dot(xq, wq_ref[0], preferred_element_type=jnp.float32) + bq_ref[0]
    k = jnp.dot(xkv, wk_ref[0], preferred_element_type=jnp.float32) + bk_ref[0]
    v = jnp.dot(xkv, wv_ref[0], preferred_element_type=jnp.float32) + bv_ref[0]
    s = jax.lax.dot_general(
        q, k, (((1,), (1,)), ((), ())), preferred_element_type=jnp.float32
    ) * _ATTN_SCALE
    m = jnp.max(s, axis=-1, keepdims=True)
    p = jnp.exp(s - m)
    p = p / jnp.sum(p, axis=-1, keepdims=True)
    oh = jnp.dot(p, v, preferred_element_type=jnp.float32)
    contrib = jnp.dot(oh, wo_ref[0], preferred_element_type=jnp.float32)

    @pl.when(h == 0)
    def _():
        acc_ref[...] = contrib

    @pl.when(h != 0)
    def _():
        acc_ref[...] += contrib

    @pl.when(h == H - 1)
    def _():
        t = acc_ref[...] + bo_ref[...] + xq
        mu = jnp.mean(t, axis=-1, keepdims=True)
        var = jnp.mean((t - mu) ** 2, axis=-1, keepdims=True)
        o_ref[...] = (t - mu) * jax.lax.rsqrt(var + 1e-5) * g_ref[...] + bb_ref[...]


def _heads_w(w):
    return w.reshape(D, H, DH).transpose(1, 0, 2)


def _mha_ln(xq, xkv, ap, lnp):
    Sq = xq.shape[0]
    Skv = xkv.shape[0]
    return pl.pallas_call(
        _mha_body,
        grid=(H,),
        in_specs=[
            pl.BlockSpec((Sq, D), lambda h: (0, 0)),
            pl.BlockSpec((Skv, D), lambda h: (0, 0)),
            pl.BlockSpec((1, D, DH), lambda h: (h, 0, 0)),
            pl.BlockSpec((1, 1, DH), lambda h: (h, 0, 0)),
            pl.BlockSpec((1, D, DH), lambda h: (h, 0, 0)),
            pl.BlockSpec((1, 1, DH), lambda h: (h, 0, 0)),
            pl.BlockSpec((1, D, DH), lambda h: (h, 0, 0)),
            pl.BlockSpec((1, 1, DH), lambda h: (h, 0, 0)),
            pl.BlockSpec((1, DH, D), lambda h: (h, 0, 0)),
            pl.BlockSpec((1, D), lambda h: (0, 0)),
            pl.BlockSpec((1, D), lambda h: (0, 0)),
            pl.BlockSpec((1, D), lambda h: (0, 0)),
        ],
        out_specs=pl.BlockSpec((Sq, D), lambda h: (0, 0)),
        out_shape=jax.ShapeDtypeStruct((Sq, D), jnp.float32),
        scratch_shapes=[pltpu.VMEM((Sq, D), jnp.float32)],
    )(
        xq, xkv,
        _heads_w(ap['wq']), ap['bq'].reshape(H, 1, DH),
        _heads_w(ap['wk']), ap['bk'].reshape(H, 1, DH),
        _heads_w(ap['wv']), ap['bv'].reshape(H, 1, DH),
        ap['wo'].reshape(H, DH, D), ap['bo'].reshape(1, D),
        lnp['g'].reshape(1, D), lnp['b'].reshape(1, D),
    )


def _moe_body(x_ref, rw_ref, rb_ref, w1_ref, b1_ref, w2_ref, b2_ref,
              g_ref, bb_ref, o_ref, acc_ref):
    e = pl.program_id(0)
    x = x_ref[...]
    logits = (
        jnp.dot(x, rw_ref[...], preferred_element_type=jnp.float32) + rb_ref[...]
    )
    mx = jnp.max(logits, axis=-1, keepdims=True)
    ex = jnp.exp(logits - mx)
    probs = ex / jnp.sum(ex, axis=-1, keepdims=True)
    cols = jax.lax.broadcasted_iota(jnp.int32, probs.shape, 1)
    m1 = jnp.max(probs, axis=-1, keepdims=True)
    i1 = jnp.min(jnp.where(probs == m1, cols, NE), axis=-1, keepdims=True)
    masked = jnp.where(cols == i1, -1.0, probs)
    m2 = jnp.max(masked, axis=-1, keepdims=True)
    i2 = jnp.min(jnp.where(masked == m2, cols, NE), axis=-1, keepdims=True)
    p_e = jnp.sum(jnp.where(cols == e, probs, 0.0), axis=-1, keepdims=True)
    sel = (i1 == e) | (i2 == e)
    we = jnp.where(sel, p_e, 0.0) / (m1 + m2)

    h = jnp.maximum(
        jnp.dot(x, w1_ref[0], preferred_element_type=jnp.float32) + b1_ref[0],
        0.0,
    )
    y = jnp.dot(h, w2_ref[0], preferred_element_type=jnp.float32) + b2_ref[0]
    contrib = we * y

    @pl.when(e == 0)
    def _():
        acc_ref[...] = contrib

    @pl.when(e != 0)
    def _():
        acc_ref[...] += contrib

    @pl.when(e == NE - 1)
    def _():
        t = x + acc_ref[...]
        mu = jnp.mean(t, axis=-1, keepdims=True)
        var = jnp.mean((t - mu) ** 2, axis=-1, keepdims=True)
        o_ref[...] = (t - mu) * jax.lax.rsqrt(var + 1e-5) * g_ref[...] + bb_ref[...]


def _moe_ln(x, mp, g, beta):
    T = x.shape[0]
    return pl.pallas_call(
        _moe_body,
        grid=(NE,),
        in_specs=[
            pl.BlockSpec((T, D), lambda e: (0, 0)),
            pl.BlockSpec((D, NE), lambda e: (0, 0)),
            pl.BlockSpec((1, NE), lambda e: (0, 0)),
            pl.BlockSpec((1, D, DFF), lambda e: (e, 0, 0)),
            pl.BlockSpec((1, 1, DFF), lambda e: (e, 0, 0)),
            pl.BlockSpec((1, DFF, D), lambda e: (e, 0, 0)),
            pl.BlockSpec((1, 1, D), lambda e: (e, 0, 0)),
            pl.BlockSpec((1, D), lambda e: (0, 0)),
            pl.BlockSpec((1, D), lambda e: (0, 0)),
        ],
        out_specs=pl.BlockSpec((T, D), lambda e: (0, 0)),
        out_shape=jax.ShapeDtypeStruct((T, D), jnp.float32),
        scratch_shapes=[pltpu.VMEM((T, D), jnp.float32)],
    )(x, mp['rw'], mp['rb'].reshape(1, NE), mp['w1'], mp['b1'].reshape(NE, 1, DFF),
      mp['w2'], mp['b2'].reshape(NE, 1, D), g.reshape(1, D), beta.reshape(1, D))


def kernel(src, tgt, params):
    p = params
    src_i = src[0]
    tgt_i = tgt[0]

    x = _embed(p['enc_emb'], src_i, _pe_table(src_i.shape[0]))
    for lp in p['enc']:
        x = _mha_ln(x, x, lp['sa'], lp['ln1'])
        x = _moe_ln(x, lp['moe'], lp['ln2']['g'], lp['ln2']['b'])
    mem = x

    y = _embed(p['dec_emb'], tgt_i, _pe_table(tgt_i.shape[0]))
    for lp in p['dec']:
        y = _mha_ln(y, y, lp['sa'], lp['ln1'])
        y = _mha_ln(y, mem, lp['ca'], lp['ln2'])
        y = _moe_ln(y, lp['moe'], lp['ln3']['g'], lp['ln3']['b'])

    logits = _mm_bias_blocked(y, p['out_w'], p['out_b'], 3200)
    return logits[None]
